# baseline (speedup 1.0000x reference)
import functools

import jax
import jax.numpy as jnp
from jax import lax
from jax.experimental import pallas as pl
from jax.experimental.pallas import tpu as pltpu
from jax.experimental.pallas import tpu_sc as plsc

B = 4096
D = 64
P = 4
K = 256
DC = 16
NI = 100000
NC = 2
NS = 16
NW = NC * NS
BPW = B // NW
NBUF = 8

_MESH = plsc.VectorSubcoreMesh(core_axis_name="c", subcore_axis_name="s")


NB_T = 8192


def _prep_body(muT_ref, lvT_ref, out_ref):
    eye = (lax.broadcasted_iota(jnp.int32, (D, D), 0)
           == lax.broadcasted_iota(jnp.int32, (D, D), 1)).astype(jnp.float32)
    out_ref[:, 0:D] = lax.dot_general(
        muT_ref[...], eye, (((0,), (0,)), ((), ())),
        precision=lax.Precision.HIGHEST)
    out_ref[:, D:2 * D] = lax.dot_general(
        lvT_ref[...], eye, (((0,), (0,)), ((), ())),
        precision=lax.Precision.HIGHEST)


def _tc_prep(muT, lvT):
    return pl.pallas_call(
        _prep_body,
        grid=((NI + NB_T - 1) // NB_T,),
        in_specs=[
            pl.BlockSpec((D, NB_T), lambda i: (0, i)),
            pl.BlockSpec((D, NB_T), lambda i: (0, i)),
        ],
        out_specs=pl.BlockSpec((NB_T, 2 * D), lambda i: (i, 0)),
        out_shape=jax.ShapeDtypeStruct((NI, 2 * D), jnp.float32),
    )(muT, lvT)


@functools.partial(
    pl.kernel,
    mesh=_MESH,
    compiler_params=pltpu.CompilerParams(needs_layout_passes=False),
    out_type=[
        jax.ShapeDtypeStruct((D, B), jnp.float32),
        jax.ShapeDtypeStruct((B, 2 * D), jnp.float32),
    ],
    scratch_types=[
        pltpu.VMEM((BPW,), jnp.int32),
        pltpu.VMEM((BPW,), jnp.int32),
        pltpu.VMEM((NBUF, D, 128), jnp.float32),
        pltpu.VMEM((D, BPW), jnp.float32),
        pltpu.VMEM((BPW, 2 * D), jnp.float32),
        pltpu.SemaphoreType.DMA,
    ] + [pltpu.SemaphoreType.DMA] * NBUF,
)
def _sc_gather(uid_hbm, iid_hbm, utT_hbm, mulv_hbm, uT_out, ms_out,
               uid_v, iid_v, ring_v, u_c, ms_v, sem_g, *sems):
    wid = lax.axis_index("s") * NC + lax.axis_index("c")
    base = wid * BPW
    pltpu.sync_copy(uid_hbm.at[pl.ds(base, BPW)], uid_v)
    pltpu.sync_copy(iid_hbm.at[pl.ds(base, BPW)], iid_v)
    cp_items = pltpu.async_copy(mulv_hbm.at[iid_v], ms_v, sem_g)
    lane = lax.iota(jnp.int32, 16)

    def fire(ru, slot):
        start = pl.multiple_of((ru >> 7) * 128, 128)
        for b in range(NBUF):
            @pl.when(slot == b)
            def _():
                pltpu.async_copy(utT_hbm.at[:, pl.ds(start, 128)],
                                 ring_v.at[b], sems[b])

    def extract(ru, slot, col):
        rc = jnp.bitwise_and(ru, 127)
        for b in range(NBUF):
            @pl.when(slot == b)
            def _():
                pltpu.make_async_copy(utT_hbm.at[:, pl.ds(0, 128)],
                                      ring_v.at[b], sems[b]).wait()
                for c4 in range(D // 16):
                    d_idx = c4 * 16 + lane
                    vals = plsc.load_gather(
                        ring_v.at[b], [d_idx, jnp.full((16,), rc, jnp.int32)])
                    plsc.store_scatter(
                        u_c, [d_idx, jnp.full((16,), col, jnp.int32)], vals)

    def sb(s, carry):
        ub = uid_v[pl.ds(pl.multiple_of(s * 16, 16), 16)]
        for j in range(NBUF):
            fire(ub[j], j)
        for j in range(NBUF):
            extract(ub[j], j, s * 16 + j)
            fire(ub[j + NBUF], j)
        for j in range(NBUF):
            extract(ub[j + NBUF], j, s * 16 + j + NBUF)
        return carry

    lax.fori_loop(0, BPW // 16, sb, 0)
    pltpu.sync_copy(u_c, uT_out.at[:, pl.ds(base, BPW)])
    cp_items.wait()
    pltpu.sync_copy(ms_v, ms_out.at[pl.ds(base, BPW)])


BB = 1024


def _dense_body(uT_ref, c_ref, g_ref, ms_ref, eps_ref, out_ref):
    lane_k = lax.broadcasted_iota(jnp.int32, (BB, K), 1)
    acc = None
    for p in range(P):
        uTp = uT_ref[p * DC:(p + 1) * DC, :]
        cp = c_ref[p]
        uc = lax.dot_general(uTp, cp, (((0,), (1,)), ((), ())),
                             precision=lax.Precision.HIGHEST)
        u2 = jnp.sum(uTp * uTp, axis=0)[:, None]
        c2 = jnp.sum(cp * cp, axis=1)[None, :]
        d2 = jnp.maximum(u2 - 2.0 * uc + c2, 0.0) + 1e-12
        score = g_ref[:, p, :] - jnp.sqrt(d2)
        am = jnp.argmax(score, axis=-1).astype(jnp.int32)
        sl = slice(p * DC, (p + 1) * DC)
        iv_p = (ms_ref[:, p * DC:(p + 1) * DC]
                + eps_ref[:, sl] * jnp.exp(0.5 * ms_ref[:, D + p * DC:
                                                        D + (p + 1) * DC]))
        w = lax.dot_general(iv_p, cp, (((1,), (1,)), ((), ())),
                            precision=lax.Precision.HIGHEST)
        part = jnp.sum(jnp.where(lane_k == am[:, None], w, 0.0), axis=1)
        acc = part if acc is None else acc + part
    out_ref[...] = acc


def _tc_dense(uT, centroids, gumbel, ms, eps):
    return pl.pallas_call(
        _dense_body,
        grid=(B // BB,),
        in_specs=[
            pl.BlockSpec((D, BB), lambda i: (0, i)),
            pl.BlockSpec((P, K, DC), lambda i: (0, 0, 0)),
            pl.BlockSpec((BB, P, K), lambda i: (i, 0, 0)),
            pl.BlockSpec((BB, 2 * D), lambda i: (i, 0)),
            pl.BlockSpec((BB, D), lambda i: (i, 0)),
        ],
        out_specs=pl.BlockSpec((BB,), lambda i: (i,)),
        out_shape=jax.ShapeDtypeStruct((B,), jnp.float32),
    )(uT, centroids, gumbel, ms, eps)


def kernel(user_id, item_id, user_table, centroids, item_mu, item_logvar,
           eps, gumbel):
    mulv = _tc_prep(item_mu.T, item_logvar.T)
    uT, ms = _sc_gather(user_id, item_id, user_table.T, mulv)
    return _tc_dense(uT, centroids, gumbel, ms, eps)


# speedup vs baseline: 1.3102x; 1.3102x over previous
import functools

import jax
import jax.numpy as jnp
from jax import lax
from jax.experimental import pallas as pl
from jax.experimental.pallas import tpu as pltpu
from jax.experimental.pallas import tpu_sc as plsc

B = 4096
D = 64
P = 4
K = 256
DC = 16
NI = 100000
NC = 2
NS = 16
NW = NC * NS
BPW = B // NW
NBUF = 8

_MESH = plsc.VectorSubcoreMesh(core_axis_name="c", subcore_axis_name="s")


NB_T = 8192


def _prep_body(muT_ref, lvT_ref, out_ref):
    out_ref[:, 0:D] = muT_ref[...].T
    out_ref[:, D:2 * D] = lvT_ref[...].T


def _tc_prep(muT, lvT):
    return pl.pallas_call(
        _prep_body,
        grid=((NI + NB_T - 1) // NB_T,),
        in_specs=[
            pl.BlockSpec((D, NB_T), lambda i: (0, i)),
            pl.BlockSpec((D, NB_T), lambda i: (0, i)),
        ],
        out_specs=pl.BlockSpec((NB_T, 2 * D), lambda i: (i, 0)),
        out_shape=jax.ShapeDtypeStruct((NI, 2 * D), jnp.float32),
    )(muT, lvT)


@functools.partial(
    pl.kernel,
    mesh=_MESH,
    compiler_params=pltpu.CompilerParams(needs_layout_passes=False),
    out_type=[
        jax.ShapeDtypeStruct((D, B), jnp.float32),
        jax.ShapeDtypeStruct((B, 2 * D), jnp.float32),
    ],
    scratch_types=[
        pltpu.VMEM((BPW,), jnp.int32),
        pltpu.VMEM((BPW,), jnp.int32),
        pltpu.VMEM((NBUF, D, 128), jnp.float32),
        pltpu.VMEM((D, BPW), jnp.float32),
        pltpu.VMEM((BPW, 2 * D), jnp.float32),
        pltpu.SemaphoreType.DMA,
    ] + [pltpu.SemaphoreType.DMA] * NBUF,
)
def _sc_gather(uid_hbm, iid_hbm, utT_hbm, mulv_hbm, uT_out, ms_out,
               uid_v, iid_v, ring_v, u_c, ms_v, sem_g, *sems):
    wid = lax.axis_index("s") * NC + lax.axis_index("c")
    base = wid * BPW
    pltpu.sync_copy(uid_hbm.at[pl.ds(base, BPW)], uid_v)
    pltpu.sync_copy(iid_hbm.at[pl.ds(base, BPW)], iid_v)
    cp_items = pltpu.async_copy(mulv_hbm.at[iid_v], ms_v, sem_g)
    lane = lax.iota(jnp.int32, 16)

    def fire(ru, slot):
        start = pl.multiple_of((ru >> 7) * 128, 128)
        for b in range(NBUF):
            @pl.when(slot == b)
            def _():
                pltpu.async_copy(utT_hbm.at[:, pl.ds(start, 128)],
                                 ring_v.at[b], sems[b])

    def extract(ru, slot, col):
        rc = jnp.bitwise_and(ru, 127)
        for b in range(NBUF):
            @pl.when(slot == b)
            def _():
                pltpu.make_async_copy(utT_hbm.at[:, pl.ds(0, 128)],
                                      ring_v.at[b], sems[b]).wait()
                for c4 in range(D // 16):
                    d_idx = c4 * 16 + lane
                    vals = plsc.load_gather(
                        ring_v.at[b], [d_idx, jnp.full((16,), rc, jnp.int32)])
                    plsc.store_scatter(
                        u_c, [d_idx, jnp.full((16,), col, jnp.int32)], vals)

    def sb(s, carry):
        ub = uid_v[pl.ds(pl.multiple_of(s * 16, 16), 16)]
        for j in range(NBUF):
            fire(ub[j], j)
        for j in range(NBUF):
            extract(ub[j], j, s * 16 + j)
            fire(ub[j + NBUF], j)
        for j in range(NBUF):
            extract(ub[j + NBUF], j, s * 16 + j + NBUF)
        return carry

    lax.fori_loop(0, BPW // 16, sb, 0)
    pltpu.sync_copy(u_c, uT_out.at[:, pl.ds(base, BPW)])
    cp_items.wait()
    pltpu.sync_copy(ms_v, ms_out.at[pl.ds(base, BPW)])


BB = 1024


def _dense_body(uT_ref, c_ref, g_ref, ms_ref, eps_ref, out_ref):
    lane_k = lax.broadcasted_iota(jnp.int32, (BB, K), 1)
    acc = None
    for p in range(P):
        uTp = uT_ref[p * DC:(p + 1) * DC, :]
        cp = c_ref[p]
        uc = lax.dot_general(uTp, cp, (((0,), (1,)), ((), ())),
                             precision=lax.Precision.HIGHEST)
        u2 = jnp.sum(uTp * uTp, axis=0)[:, None]
        c2 = jnp.sum(cp * cp, axis=1)[None, :]
        d2 = jnp.maximum(u2 - 2.0 * uc + c2, 0.0) + 1e-12
        score = g_ref[:, p, :] - jnp.sqrt(d2)
        am = jnp.argmax(score, axis=-1).astype(jnp.int32)
        sl = slice(p * DC, (p + 1) * DC)
        iv_p = (ms_ref[:, p * DC:(p + 1) * DC]
                + eps_ref[:, sl] * jnp.exp(0.5 * ms_ref[:, D + p * DC:
                                                        D + (p + 1) * DC]))
        w = lax.dot_general(iv_p, cp, (((1,), (1,)), ((), ())),
                            precision=lax.Precision.HIGHEST)
        part = jnp.sum(jnp.where(lane_k == am[:, None], w, 0.0), axis=1)
        acc = part if acc is None else acc + part
    out_ref[...] = acc


def _tc_dense(uT, centroids, gumbel, ms, eps):
    return pl.pallas_call(
        _dense_body,
        grid=(B // BB,),
        in_specs=[
            pl.BlockSpec((D, BB), lambda i: (0, i)),
            pl.BlockSpec((P, K, DC), lambda i: (0, 0, 0)),
            pl.BlockSpec((BB, P, K), lambda i: (i, 0, 0)),
            pl.BlockSpec((BB, 2 * D), lambda i: (i, 0)),
            pl.BlockSpec((BB, D), lambda i: (i, 0)),
        ],
        out_specs=pl.BlockSpec((BB,), lambda i: (i,)),
        out_shape=jax.ShapeDtypeStruct((B,), jnp.float32),
    )(uT, centroids, gumbel, ms, eps)


def kernel(user_id, item_id, user_table, centroids, item_mu, item_logvar,
           eps, gumbel):
    mulv = _tc_prep(item_mu.T, item_logvar.T)
    uT, ms = _sc_gather(user_id, item_id, user_table.T, mulv)
    return _tc_dense(uT, centroids, gumbel, ms, eps)
